# baseline (speedup 1.0000x reference)
import functools

import jax
import jax.numpy as jnp
from jax import lax
from jax.experimental import pallas as pl
from jax.experimental.pallas import tpu as pltpu
from jax.experimental.pallas import tpu_sc as plsc

N = 10000
E = 320000
D = 128
H = 128

NUM_CORES = 2
NUM_TILES = 16

EDGES_PER_TILE = E // NUM_TILES
CHUNK = 100
N_CHUNKS = 200
EDGES_PT_PAD = N_CHUNKS * CHUNK
PAD_PT = EDGES_PT_PAD - EDGES_PER_TILE
IDXBLK = 40
N_BLOCKS = N_CHUNKS // IDXBLK
NBUF = 3
N_QUADS = IDXBLK // NBUF
TAIL = IDXBLK - N_QUADS * NBUF
ROW_CHUNK = 80
N_ROW_CHUNKS = N // ROW_CHUNK
ROW_ITERS = -(-N_ROW_CHUNKS // NUM_TILES)


def _sc_pool_body(h_hbm, src_hbm, dst_hbm, out_hbm, acc, sidx, didx, *bufs):
    rows = bufs[:NBUF]
    sems = bufs[NBUF:2 * NBUF]
    ssems = bufs[2 * NBUF:]
    c = lax.axis_index("c")
    s = lax.axis_index("s")
    cN = c * N
    stage = rows[0].at[0:ROW_CHUNK]

    def seed_step(j, carry):
        cid = j * NUM_TILES + s

        @pl.when(cid < N_ROW_CHUNKS)
        def _():
            r = cid * ROW_CHUNK
            pltpu.sync_copy(h_hbm.at[pl.ds(cN + r, ROW_CHUNK)], stage)
            pltpu.sync_copy(stage, acc.at[pl.ds(r, ROW_CHUNK)])

        return carry

    lax.fori_loop(0, ROW_ITERS, seed_step, 0)
    plsc.subcore_barrier()

    def block_body(b, carry):
        pltpu.sync_copy(src_hbm.at[c, s, pl.ds(b * IDXBLK, IDXBLK)], sidx)
        pltpu.sync_copy(dst_hbm.at[s, pl.ds(b * IDXBLK, IDXBLK)], didx)
        for u in range(NBUF - 1):
            pltpu.async_copy(h_hbm.at[sidx.at[u]], rows[u], sems[u])

        def edge_quad(i, inner):
            for u in range(NBUF):
                j = i * NBUF + u
                nxt = (u + NBUF - 1) % NBUF

                @pl.when(jnp.logical_and(j + NBUF - 1 < IDXBLK, j >= 1))
                def _():
                    pltpu.make_async_copy(rows[nxt], acc.at[didx.at[j]],
                                          ssems[nxt]).wait()

                @pl.when(j + NBUF - 1 < IDXBLK)
                def _():
                    pltpu.async_copy(h_hbm.at[sidx.at[j + NBUF - 1]],
                                     rows[nxt], sems[nxt])

                pltpu.make_async_copy(h_hbm.at[sidx.at[j]], rows[u],
                                      sems[u]).wait()
                pltpu.async_copy(rows[u], acc.at[didx.at[j]], ssems[u],
                                 add=True)
            return inner

        lax.fori_loop(0, N_QUADS, edge_quad, 0)
        for t in range(TAIL):
            j = N_QUADS * NBUF + t
            u = j % NBUF
            pltpu.make_async_copy(h_hbm.at[sidx.at[j]], rows[u],
                                  sems[u]).wait()
            pltpu.async_copy(rows[u], acc.at[didx.at[j]], ssems[u],
                             add=True)
        for u in range(NBUF):
            pltpu.make_async_copy(rows[u], acc.at[didx.at[u]],
                                  ssems[u]).wait()
        return carry

    lax.fori_loop(0, N_BLOCKS, block_body, 0)
    plsc.subcore_barrier()

    def out_step(j, carry):
        cid = j * NUM_TILES + s

        @pl.when(cid < N_ROW_CHUNKS)
        def _():
            r = cid * ROW_CHUNK
            pltpu.sync_copy(acc.at[pl.ds(r, ROW_CHUNK)], stage)
            pltpu.sync_copy(stage, out_hbm.at[pl.ds(cN + r, ROW_CHUNK)])

        return carry

    lax.fori_loop(0, ROW_ITERS, out_step, 0)


@functools.cache
def _make_sc_pool():
    return pl.kernel(
        _sc_pool_body,
        out_type=jax.ShapeDtypeStruct((2 * N, H), jnp.float32),
        mesh=plsc.VectorSubcoreMesh(core_axis_name="c", subcore_axis_name="s"),
        scratch_types=[
            pltpu.VMEM_SHARED((N + 64, H), jnp.float32),
            pltpu.VMEM((IDXBLK, CHUNK), jnp.int32),
            pltpu.VMEM((IDXBLK, CHUNK), jnp.int32),
        ] + [pltpu.VMEM((CHUNK, H), jnp.float32)] * NBUF
          + [pltpu.SemaphoreType.DMA] * (2 * NBUF),
    )


def _bn_relu(y, g, b):
    m = jnp.mean(y, axis=0, keepdims=True)
    v = jnp.mean((y - m) * (y - m), axis=0, keepdims=True)
    return jnp.maximum(g * (y - m) * lax.rsqrt(v + 1e-5) + b, 0.0)


def _dense_body(x_ref, w1_ref, b1_ref, g1_ref, be1_ref,
                w2_ref, b2_ref, g2_ref, be2_ref, out_ref):
    w1 = w1_ref[...]
    w2 = w2_ref[...]
    b1 = b1_ref[...]
    g1 = g1_ref[...]
    be1 = be1_ref[...]
    b2 = b2_ref[...]
    g2 = g2_ref[...]
    be2 = be2_ref[...]
    for p in range(2):
        x = x_ref[p * N:(p + 1) * N, :]
        y = jnp.dot(x, w1, preferred_element_type=jnp.float32,
                    precision=lax.Precision.HIGHEST) + b1
        h1 = _bn_relu(y, g1, be1)
        y2 = jnp.dot(h1, w2, preferred_element_type=jnp.float32,
                     precision=lax.Precision.HIGHEST) + b2
        out_ref[p * N:(p + 1) * N, :] = _bn_relu(y2, g2, be2)


def _dense_layer(x, w1, b1, g1, be1, w2, b2, g2, be2):
    return pl.pallas_call(
        _dense_body,
        out_shape=jax.ShapeDtypeStruct((2 * N, H), jnp.float32),
    )(x, w1, b1.reshape(1, H), g1.reshape(1, H), be1.reshape(1, H),
      w2, b2.reshape(1, H), g2.reshape(1, H), be2.reshape(1, H))


def _bce(logit, label):
    return (jnp.maximum(logit, 0.0) - logit * label
            + jnp.log1p(jnp.exp(-jnp.abs(logit))))


def _loss_body(h_ref, msk_ref, sb1_ref, sb2_ref, lbl1_ref, lbl2_ref,
               dw_ref, db_ref, out_ref):
    h1 = h_ref[0:N, :]
    h2 = h_ref[N:2 * N, :]
    msk = msk_ref[...]
    c = jnp.dot(msk, h1, preferred_element_type=jnp.float32,
                precision=lax.Precision.HIGHEST) / jnp.sum(msk)
    c = 1.0 / (1.0 + jnp.exp(-c))
    cw = jnp.dot(c, dw_ref[...], preferred_element_type=jnp.float32,
                 precision=lax.Precision.HIGHEST)
    db = db_ref[0, 0]
    s1 = jnp.sum(h1 * cw, axis=1, keepdims=True) + db + sb1_ref[...]
    s2 = jnp.sum(h2 * cw, axis=1, keepdims=True) + db + sb2_ref[...]
    tot = jnp.sum(_bce(s1, lbl1_ref[...])) + jnp.sum(_bce(s2, lbl2_ref[...]))
    out_ref[...] = jnp.reshape(tot / (2.0 * N), (1, 1))


def _loss(h, msk, sb1, sb2, lbl, disc_w, disc_b):
    out = pl.pallas_call(
        _loss_body,
        out_shape=jax.ShapeDtypeStruct((1, 1), jnp.float32),
    )(h, msk, sb1.reshape(N, 1), sb2.reshape(N, 1),
      lbl[:, :N].reshape(N, 1), lbl[:, N:].reshape(N, 1),
      disc_w, disc_b.reshape(1, 1))
    return out.reshape(())


def kernel(seq1, seq2, adj, msk, samp_bias1, samp_bias2, lbl,
           gin0_W1, gin0_b1, gin0_g1, gin0_be1, gin0_W2, gin0_b2, gin0_g2, gin0_be2,
           gin1_W1, gin1_b1, gin1_g1, gin1_be1, gin1_W2, gin1_b2, gin1_g2, gin1_be2,
           disc_W, disc_b):
    src = adj[0]
    dst = adj[1]
    src_r = jnp.concatenate(
        [src.reshape(NUM_TILES, EDGES_PER_TILE),
         jnp.zeros((NUM_TILES, PAD_PT), jnp.int32)], axis=1)
    junk = N + (jnp.arange(PAD_PT, dtype=jnp.int32) % 64)
    dst_r = jnp.concatenate(
        [dst.reshape(NUM_TILES, EDGES_PER_TILE),
         jnp.broadcast_to(junk, (NUM_TILES, PAD_PT))], axis=1)
    src2 = jnp.stack([src_r, src_r + N]).reshape(2, NUM_TILES, N_CHUNKS, CHUNK)
    dst2 = dst_r.reshape(NUM_TILES, N_CHUNKS, CHUNK)
    h = jnp.concatenate([seq1, seq2], axis=0)
    layers = (
        (gin0_W1, gin0_b1, gin0_g1, gin0_be1, gin0_W2, gin0_b2, gin0_g2, gin0_be2),
        (gin1_W1, gin1_b1, gin1_g1, gin1_be1, gin1_W2, gin1_b2, gin1_g2, gin1_be2),
    )
    sc_pool = _make_sc_pool()
    for lw in layers:
        pooled = sc_pool(h, src2, dst2)
        h = _dense_layer(pooled, *lw)
    return _loss(h, msk, samp_bias1, samp_bias2, lbl, disc_W, disc_b)


# speedup vs baseline: 1.0085x; 1.0085x over previous
import functools

import jax
import jax.numpy as jnp
from jax import lax
from jax.experimental import pallas as pl
from jax.experimental.pallas import tpu as pltpu
from jax.experimental.pallas import tpu_sc as plsc

N = 10000
E = 320000
D = 128
H = 128

NUM_CORES = 2
NUM_TILES = 16

EDGES_PER_TILE = E // NUM_TILES
CHUNK = 100
N_CHUNKS = 200
EDGES_PT_PAD = N_CHUNKS * CHUNK
PAD_PT = EDGES_PT_PAD - EDGES_PER_TILE
IDXBLK = 40
N_BLOCKS = N_CHUNKS // IDXBLK
NBUF = 3
N_QUADS = IDXBLK // NBUF
TAIL = IDXBLK - N_QUADS * NBUF
ROW_CHUNK = 80
N_ROW_CHUNKS = N // ROW_CHUNK
ROW_ITERS = -(-N_ROW_CHUNKS // NUM_TILES)


def _sc_pool_body(h_hbm, src_hbm, dst_hbm, out_hbm, acc, sidx, didx, *bufs):
    rows = bufs[:NBUF]
    sems = bufs[NBUF:2 * NBUF]
    ssems = bufs[2 * NBUF:]
    c = lax.axis_index("c")
    s = lax.axis_index("s")
    cN = c * N
    stage = rows[0].at[0:ROW_CHUNK]

    def seed_step(j, carry):
        cid = j * NUM_TILES + s

        @pl.when(cid < N_ROW_CHUNKS)
        def _():
            r = cid * ROW_CHUNK
            pltpu.sync_copy(h_hbm.at[pl.ds(cN + r, ROW_CHUNK)], stage)
            pltpu.sync_copy(stage, acc.at[pl.ds(r, ROW_CHUNK)])

        return carry

    lax.fori_loop(0, ROW_ITERS, seed_step, 0)
    plsc.subcore_barrier()

    def block_body(b, carry):
        pltpu.sync_copy(src_hbm.at[c, s, pl.ds(b * IDXBLK, IDXBLK)], sidx)
        pltpu.sync_copy(dst_hbm.at[s, pl.ds(b * IDXBLK, IDXBLK)], didx)
        for u in range(NBUF - 1):
            pltpu.async_copy(h_hbm.at[sidx.at[u]], rows[u], sems[u])

        def edge_quad(i, inner):
            for u in range(NBUF):
                j = i * NBUF + u
                nxt = (u + NBUF - 1) % NBUF

                @pl.when(jnp.logical_and(j + NBUF - 1 < IDXBLK, j >= 1))
                def _():
                    pltpu.make_async_copy(rows[nxt], acc.at[didx.at[j]],
                                          ssems[nxt]).wait()

                @pl.when(j + NBUF - 1 < IDXBLK)
                def _():
                    pltpu.async_copy(h_hbm.at[sidx.at[j + NBUF - 1]],
                                     rows[nxt], sems[nxt])

                pltpu.make_async_copy(h_hbm.at[sidx.at[j]], rows[u],
                                      sems[u]).wait()
                pltpu.async_copy(rows[u], acc.at[didx.at[j]], ssems[u],
                                 add=True)
            return inner

        lax.fori_loop(0, N_QUADS, edge_quad, 0)
        for t in range(TAIL):
            j = N_QUADS * NBUF + t
            u = j % NBUF
            pltpu.make_async_copy(h_hbm.at[sidx.at[j]], rows[u],
                                  sems[u]).wait()
            pltpu.async_copy(rows[u], acc.at[didx.at[j]], ssems[u],
                             add=True)
        for u in range(NBUF):
            pltpu.make_async_copy(rows[u], acc.at[didx.at[u]],
                                  ssems[u]).wait()
        return carry

    lax.fori_loop(0, N_BLOCKS, block_body, 0)
    plsc.subcore_barrier()

    def out_step(j, carry):
        cid = j * NUM_TILES + s

        @pl.when(cid < N_ROW_CHUNKS)
        def _():
            r = cid * ROW_CHUNK
            pltpu.sync_copy(acc.at[pl.ds(r, ROW_CHUNK)], stage)
            pltpu.sync_copy(stage, out_hbm.at[pl.ds(cN + r, ROW_CHUNK)])

        return carry

    lax.fori_loop(0, ROW_ITERS, out_step, 0)


@functools.cache
def _make_sc_pool():
    return pl.kernel(
        _sc_pool_body,
        out_type=jax.ShapeDtypeStruct((2 * N, H), jnp.float32),
        mesh=plsc.VectorSubcoreMesh(core_axis_name="c", subcore_axis_name="s"),
        scratch_types=[
            pltpu.VMEM_SHARED((N + 64, H), jnp.float32),
            pltpu.VMEM((IDXBLK, CHUNK), jnp.int32),
            pltpu.VMEM((IDXBLK, CHUNK), jnp.int32),
        ] + [pltpu.VMEM((CHUNK, H), jnp.float32)] * NBUF
          + [pltpu.SemaphoreType.DMA] * (2 * NBUF),
    )


def _bn_relu(y, g, b):
    m = jnp.mean(y, axis=0, keepdims=True)
    v = jnp.mean((y - m) * (y - m), axis=0, keepdims=True)
    return jnp.maximum(g * (y - m) * lax.rsqrt(v + 1e-5) + b, 0.0)


def _dense_body(x_ref, w1_ref, b1_ref, g1_ref, be1_ref,
                w2_ref, b2_ref, g2_ref, be2_ref, out_ref):
    w1 = w1_ref[...]
    w2 = w2_ref[...]
    b1 = b1_ref[...]
    g1 = g1_ref[...]
    be1 = be1_ref[...]
    b2 = b2_ref[...]
    g2 = g2_ref[...]
    be2 = be2_ref[...]
    for p in range(2):
        x = x_ref[p * N:(p + 1) * N, :]
        y = jnp.dot(x, w1, preferred_element_type=jnp.float32,
                    precision=lax.Precision.HIGHEST) + b1
        h1 = _bn_relu(y, g1, be1)
        y2 = jnp.dot(h1, w2, preferred_element_type=jnp.float32,
                     precision=lax.Precision.HIGHEST) + b2
        out_ref[p * N:(p + 1) * N, :] = _bn_relu(y2, g2, be2)


def _dense_layer(x, w1, b1, g1, be1, w2, b2, g2, be2):
    return pl.pallas_call(
        _dense_body,
        out_shape=jax.ShapeDtypeStruct((2 * N, H), jnp.float32),
    )(x, w1, b1.reshape(1, H), g1.reshape(1, H), be1.reshape(1, H),
      w2, b2.reshape(1, H), g2.reshape(1, H), be2.reshape(1, H))


def _bce(logit, label):
    return (jnp.maximum(logit, 0.0) - logit * label
            + jnp.log1p(jnp.exp(-jnp.abs(logit))))


def _dense2_loss_body(x_ref, w1_ref, b1_ref, g1_ref, be1_ref,
                      w2_ref, b2_ref, g2_ref, be2_ref,
                      msk_ref, sb1_ref, sb2_ref, lbl1_ref, lbl2_ref,
                      dw_ref, db_ref, out_ref):
    w1 = w1_ref[...]
    w2 = w2_ref[...]
    b1 = b1_ref[...]
    g1 = g1_ref[...]
    be1 = be1_ref[...]
    b2 = b2_ref[...]
    g2 = g2_ref[...]
    be2 = be2_ref[...]
    def pass_h(p):
        x = x_ref[p * N:(p + 1) * N, :]
        y = jnp.dot(x, w1, preferred_element_type=jnp.float32,
                    precision=lax.Precision.HIGHEST) + b1
        hh = _bn_relu(y, g1, be1)
        y2 = jnp.dot(hh, w2, preferred_element_type=jnp.float32,
                     precision=lax.Precision.HIGHEST) + b2
        return _bn_relu(y2, g2, be2)

    db = db_ref[0, 0]
    msk = msk_ref[...]
    h1 = pass_h(0)
    c = jnp.dot(msk, h1, preferred_element_type=jnp.float32,
                precision=lax.Precision.HIGHEST) / jnp.sum(msk)
    c = 1.0 / (1.0 + jnp.exp(-c))
    cw = jnp.dot(c, dw_ref[...], preferred_element_type=jnp.float32,
                 precision=lax.Precision.HIGHEST)
    s1 = jnp.sum(h1 * cw, axis=1, keepdims=True) + db + sb1_ref[...]
    tot = jnp.sum(_bce(s1, lbl1_ref[...]))
    h2 = pass_h(1)
    s2 = jnp.sum(h2 * cw, axis=1, keepdims=True) + db + sb2_ref[...]
    tot = tot + jnp.sum(_bce(s2, lbl2_ref[...]))
    out_ref[...] = jnp.reshape(tot / (2.0 * N), (1, 1))


def _dense2_loss(x, w1, b1, g1, be1, w2, b2, g2, be2,
                 msk, sb1, sb2, lbl, disc_w, disc_b):
    out = pl.pallas_call(
        _dense2_loss_body,
        out_shape=jax.ShapeDtypeStruct((1, 1), jnp.float32),
    )(x, w1, b1.reshape(1, H), g1.reshape(1, H), be1.reshape(1, H),
      w2, b2.reshape(1, H), g2.reshape(1, H), be2.reshape(1, H),
      msk, sb1.reshape(N, 1), sb2.reshape(N, 1),
      lbl[:, :N].reshape(N, 1), lbl[:, N:].reshape(N, 1),
      disc_w, disc_b.reshape(1, 1))
    return out.reshape(())


def kernel(seq1, seq2, adj, msk, samp_bias1, samp_bias2, lbl,
           gin0_W1, gin0_b1, gin0_g1, gin0_be1, gin0_W2, gin0_b2, gin0_g2, gin0_be2,
           gin1_W1, gin1_b1, gin1_g1, gin1_be1, gin1_W2, gin1_b2, gin1_g2, gin1_be2,
           disc_W, disc_b):
    src = adj[0]
    dst = adj[1]
    src_r = jnp.concatenate(
        [src.reshape(NUM_TILES, EDGES_PER_TILE),
         jnp.zeros((NUM_TILES, PAD_PT), jnp.int32)], axis=1)
    junk = N + (jnp.arange(PAD_PT, dtype=jnp.int32) % 64)
    dst_r = jnp.concatenate(
        [dst.reshape(NUM_TILES, EDGES_PER_TILE),
         jnp.broadcast_to(junk, (NUM_TILES, PAD_PT))], axis=1)
    src2 = jnp.stack([src_r, src_r + N]).reshape(2, NUM_TILES, N_CHUNKS, CHUNK)
    dst2 = dst_r.reshape(NUM_TILES, N_CHUNKS, CHUNK)
    h = jnp.concatenate([seq1, seq2], axis=0)
    layers = (
        (gin0_W1, gin0_b1, gin0_g1, gin0_be1, gin0_W2, gin0_b2, gin0_g2, gin0_be2),
        (gin1_W1, gin1_b1, gin1_g1, gin1_be1, gin1_W2, gin1_b2, gin1_g2, gin1_be2),
    )
    sc_pool = _make_sc_pool()
    pooled = sc_pool(h, src2, dst2)
    h = _dense_layer(pooled, *layers[0])
    pooled = sc_pool(h, src2, dst2)
    return _dense2_loss(pooled, *layers[1],
                        msk, samp_bias1, samp_bias2, lbl, disc_W, disc_b)


# speedup vs baseline: 1.0253x; 1.0167x over previous
import functools

import jax
import jax.numpy as jnp
from jax import lax
from jax.experimental import pallas as pl
from jax.experimental.pallas import tpu as pltpu
from jax.experimental.pallas import tpu_sc as plsc

N = 10000
E = 320000
D = 128
H = 128

NUM_CORES = 2
NUM_TILES = 16

EDGES_PER_TILE = E // NUM_TILES
CHUNK = 100
N_CHUNKS = 200
EDGES_PT_PAD = N_CHUNKS * CHUNK
PAD_PT = EDGES_PT_PAD - EDGES_PER_TILE
IDXBLK = 40
N_BLOCKS = N_CHUNKS // IDXBLK
NBUF = 3
N_QUADS = IDXBLK // NBUF
TAIL = IDXBLK - N_QUADS * NBUF
ROW_CHUNK = 80
N_ROW_CHUNKS = N // ROW_CHUNK
ROW_ITERS = -(-N_ROW_CHUNKS // NUM_TILES)


def _sc_pool_body(h_hbm, src_hbm, dst_hbm, out_hbm, acc, sidx, didx, *bufs):
    rows = bufs[:NBUF]
    sems = bufs[NBUF:2 * NBUF]
    ssems = bufs[2 * NBUF:]
    c = lax.axis_index("c")
    s = lax.axis_index("s")
    cN = c * N
    stage = rows[0].at[0:ROW_CHUNK]

    def seed_step(j, carry):
        cid = j * NUM_TILES + s

        @pl.when(cid < N_ROW_CHUNKS)
        def _():
            r = cid * ROW_CHUNK
            pltpu.sync_copy(h_hbm.at[pl.ds(cN + r, ROW_CHUNK)],
                            acc.at[pl.ds(r, ROW_CHUNK)])

        return carry

    lax.fori_loop(0, ROW_ITERS, seed_step, 0)
    plsc.subcore_barrier()

    def block_body(b, carry):
        pltpu.sync_copy(src_hbm.at[c, s, pl.ds(b * IDXBLK, IDXBLK)], sidx)
        pltpu.sync_copy(dst_hbm.at[s, pl.ds(b * IDXBLK, IDXBLK)], didx)
        for u in range(NBUF - 1):
            pltpu.async_copy(h_hbm.at[sidx.at[u]], rows[u], sems[u])

        def edge_quad(i, inner):
            for u in range(NBUF):
                j = i * NBUF + u
                nxt = (u + NBUF - 1) % NBUF

                @pl.when(jnp.logical_and(j + NBUF - 1 < IDXBLK, j >= 1))
                def _():
                    pltpu.make_async_copy(rows[nxt], acc.at[didx.at[j]],
                                          ssems[nxt]).wait()

                @pl.when(j + NBUF - 1 < IDXBLK)
                def _():
                    pltpu.async_copy(h_hbm.at[sidx.at[j + NBUF - 1]],
                                     rows[nxt], sems[nxt])

                pltpu.make_async_copy(h_hbm.at[sidx.at[j]], rows[u],
                                      sems[u]).wait()
                pltpu.async_copy(rows[u], acc.at[didx.at[j]], ssems[u],
                                 add=True)
            return inner

        lax.fori_loop(0, N_QUADS, edge_quad, 0)
        for t in range(TAIL):
            j = N_QUADS * NBUF + t
            u = j % NBUF
            pltpu.make_async_copy(h_hbm.at[sidx.at[j]], rows[u],
                                  sems[u]).wait()
            pltpu.async_copy(rows[u], acc.at[didx.at[j]], ssems[u],
                             add=True)
        for u in range(NBUF):
            pltpu.make_async_copy(rows[u], acc.at[didx.at[u]],
                                  ssems[u]).wait()
        return carry

    lax.fori_loop(0, N_BLOCKS, block_body, 0)
    plsc.subcore_barrier()

    def out_step(j, carry):
        cid = j * NUM_TILES + s

        @pl.when(cid < N_ROW_CHUNKS)
        def _():
            r = cid * ROW_CHUNK
            pltpu.sync_copy(acc.at[pl.ds(r, ROW_CHUNK)],
                            out_hbm.at[pl.ds(cN + r, ROW_CHUNK)])

        return carry

    lax.fori_loop(0, ROW_ITERS, out_step, 0)


@functools.cache
def _make_sc_pool():
    return pl.kernel(
        _sc_pool_body,
        out_type=jax.ShapeDtypeStruct((2 * N, H), jnp.float32),
        mesh=plsc.VectorSubcoreMesh(core_axis_name="c", subcore_axis_name="s"),
        scratch_types=[
            pltpu.VMEM_SHARED((N + 64, H), jnp.float32),
            pltpu.VMEM((IDXBLK, CHUNK), jnp.int32),
            pltpu.VMEM((IDXBLK, CHUNK), jnp.int32),
        ] + [pltpu.VMEM((CHUNK, H), jnp.float32)] * NBUF
          + [pltpu.SemaphoreType.DMA] * (2 * NBUF),
    )


def _bn_relu(y, g, b):
    m = jnp.mean(y, axis=0, keepdims=True)
    v = jnp.mean((y - m) * (y - m), axis=0, keepdims=True)
    return jnp.maximum(g * (y - m) * lax.rsqrt(v + 1e-5) + b, 0.0)


def _dense_body(x_ref, w1_ref, b1_ref, g1_ref, be1_ref,
                w2_ref, b2_ref, g2_ref, be2_ref, out_ref):
    w1 = w1_ref[...]
    w2 = w2_ref[...]
    b1 = b1_ref[...]
    g1 = g1_ref[...]
    be1 = be1_ref[...]
    b2 = b2_ref[...]
    g2 = g2_ref[...]
    be2 = be2_ref[...]
    for p in range(2):
        x = x_ref[p * N:(p + 1) * N, :]
        y = jnp.dot(x, w1, preferred_element_type=jnp.float32,
                    precision=lax.Precision.HIGHEST) + b1
        h1 = _bn_relu(y, g1, be1)
        y2 = jnp.dot(h1, w2, preferred_element_type=jnp.float32,
                     precision=lax.Precision.HIGHEST) + b2
        out_ref[p * N:(p + 1) * N, :] = _bn_relu(y2, g2, be2)


def _dense_layer(x, w1, b1, g1, be1, w2, b2, g2, be2):
    return pl.pallas_call(
        _dense_body,
        out_shape=jax.ShapeDtypeStruct((2 * N, H), jnp.float32),
    )(x, w1, b1.reshape(1, H), g1.reshape(1, H), be1.reshape(1, H),
      w2, b2.reshape(1, H), g2.reshape(1, H), be2.reshape(1, H))


def _bce(logit, label):
    return (jnp.maximum(logit, 0.0) - logit * label
            + jnp.log1p(jnp.exp(-jnp.abs(logit))))


def _dense2_loss_body(x_ref, w1_ref, b1_ref, g1_ref, be1_ref,
                      w2_ref, b2_ref, g2_ref, be2_ref,
                      msk_ref, sb1_ref, sb2_ref, lbl1_ref, lbl2_ref,
                      dw_ref, db_ref, out_ref):
    w1 = w1_ref[...]
    w2 = w2_ref[...]
    b1 = b1_ref[...]
    g1 = g1_ref[...]
    be1 = be1_ref[...]
    b2 = b2_ref[...]
    g2 = g2_ref[...]
    be2 = be2_ref[...]
    def pass_h(p):
        x = x_ref[p * N:(p + 1) * N, :]
        y = jnp.dot(x, w1, preferred_element_type=jnp.float32,
                    precision=lax.Precision.HIGHEST) + b1
        hh = _bn_relu(y, g1, be1)
        y2 = jnp.dot(hh, w2, preferred_element_type=jnp.float32,
                     precision=lax.Precision.HIGHEST) + b2
        return _bn_relu(y2, g2, be2)

    db = db_ref[0, 0]
    msk = msk_ref[...]
    h1 = pass_h(0)
    c = jnp.dot(msk, h1, preferred_element_type=jnp.float32,
                precision=lax.Precision.HIGHEST) / jnp.sum(msk)
    c = 1.0 / (1.0 + jnp.exp(-c))
    cw = jnp.dot(c, dw_ref[...], preferred_element_type=jnp.float32,
                 precision=lax.Precision.HIGHEST)
    s1 = jnp.sum(h1 * cw, axis=1, keepdims=True) + db + sb1_ref[...]
    tot = jnp.sum(_bce(s1, lbl1_ref[...]))
    h2 = pass_h(1)
    s2 = jnp.sum(h2 * cw, axis=1, keepdims=True) + db + sb2_ref[...]
    tot = tot + jnp.sum(_bce(s2, lbl2_ref[...]))
    out_ref[...] = jnp.reshape(tot / (2.0 * N), (1, 1))


def _dense2_loss(x, w1, b1, g1, be1, w2, b2, g2, be2,
                 msk, sb1, sb2, lbl, disc_w, disc_b):
    out = pl.pallas_call(
        _dense2_loss_body,
        out_shape=jax.ShapeDtypeStruct((1, 1), jnp.float32),
    )(x, w1, b1.reshape(1, H), g1.reshape(1, H), be1.reshape(1, H),
      w2, b2.reshape(1, H), g2.reshape(1, H), be2.reshape(1, H),
      msk, sb1.reshape(N, 1), sb2.reshape(N, 1),
      lbl[:, :N].reshape(N, 1), lbl[:, N:].reshape(N, 1),
      disc_w, disc_b.reshape(1, 1))
    return out.reshape(())


def kernel(seq1, seq2, adj, msk, samp_bias1, samp_bias2, lbl,
           gin0_W1, gin0_b1, gin0_g1, gin0_be1, gin0_W2, gin0_b2, gin0_g2, gin0_be2,
           gin1_W1, gin1_b1, gin1_g1, gin1_be1, gin1_W2, gin1_b2, gin1_g2, gin1_be2,
           disc_W, disc_b):
    src = adj[0]
    dst = adj[1]
    src_r = jnp.concatenate(
        [src.reshape(NUM_TILES, EDGES_PER_TILE),
         jnp.zeros((NUM_TILES, PAD_PT), jnp.int32)], axis=1)
    junk = N + (jnp.arange(PAD_PT, dtype=jnp.int32) % 64)
    dst_r = jnp.concatenate(
        [dst.reshape(NUM_TILES, EDGES_PER_TILE),
         jnp.broadcast_to(junk, (NUM_TILES, PAD_PT))], axis=1)
    src2 = jnp.stack([src_r, src_r + N]).reshape(2, NUM_TILES, N_CHUNKS, CHUNK)
    dst2 = dst_r.reshape(NUM_TILES, N_CHUNKS, CHUNK)
    h = jnp.concatenate([seq1, seq2], axis=0)
    layers = (
        (gin0_W1, gin0_b1, gin0_g1, gin0_be1, gin0_W2, gin0_b2, gin0_g2, gin0_be2),
        (gin1_W1, gin1_b1, gin1_g1, gin1_be1, gin1_W2, gin1_b2, gin1_g2, gin1_be2),
    )
    sc_pool = _make_sc_pool()
    pooled = sc_pool(h, src2, dst2)
    h = _dense_layer(pooled, *layers[0])
    pooled = sc_pool(h, src2, dst2)
    return _dense2_loss(pooled, *layers[1],
                        msk, samp_bias1, samp_bias2, lbl, disc_W, disc_b)
